# baseline (speedup 1.0000x reference)
import functools

import jax
import jax.numpy as jnp
from jax import lax
from jax.experimental import pallas as pl
from jax.experimental.pallas import tpu as pltpu
from jax.experimental.pallas import tpu_sc as plsc

B = 16384
D = 64
L = 16
NC = 2
NS = 16
NW = NC * NS
NWINDOWS = 7813
LASTWIN = NWINDOWS - 1
MCAP = 2048
HCAP = 64
DUMP = B
ROWS = B + 128

_mesh = plsc.VectorSubcoreMesh(core_axis_name="c", subcore_axis_name="s")


@functools.partial(
    pl.kernel,
    mesh=_mesh,
    compiler_params=pltpu.CompilerParams(
        needs_layout_passes=False, use_tc_tiling_on_sc=True),
    out_type=(jax.ShapeDtypeStruct((ROWS, 128), jnp.float32),
              jax.ShapeDtypeStruct((ROWS, 128), jnp.float32)),
    scratch_types=[
        pltpu.VMEM((B,), jnp.int32),
        pltpu.VMEM((MCAP,), jnp.int32),
        pltpu.VMEM((MCAP,), jnp.int32),
        pltpu.VMEM((HCAP,), jnp.int32),
        pltpu.VMEM((HCAP,), jnp.int32),
        pltpu.VMEM((D, 128), jnp.float32),
        pltpu.VMEM((D, 128), jnp.float32),
        pltpu.VMEM((D, 128), jnp.float32),
        pltpu.VMEM((D, 128), jnp.float32),
        pltpu.VMEM((16, 128), jnp.float32),
        pltpu.VMEM((16,), jnp.int32),
        pltpu.SemaphoreType.DMA,
        pltpu.SemaphoreType.DMA,
        pltpu.SemaphoreType.DMA,
        pltpu.SemaphoreType.DMA,
        pltpu.SemaphoreType.DMA,
    ],
)
def _sweep_sc(uids_hbm, iids_hbm, pu_hbm, pv_hbm, ug_hbm, vg_hbm,
              idb, moff, mpos, hid, hpo, w0b, w1b, w2b, w3b,
              stage, posb, sem0, sem1, sem2, sem3, ssem):
    wid = lax.axis_index("s") * NC + lax.axis_index("c")
    W0 = jnp.where(wid < 5, 245 * wid, 244 * wid + 5)
    NWIN = jnp.where(wid < 5, 245, 244)
    lanes = lax.iota(jnp.int32, L)
    lane0 = lanes == 0
    wbufs = (w0b, w1b, w2b, w3b)
    wsems = (sem0, sem1, sem2, sem3)

    zero16 = jnp.zeros((L,), jnp.float32)
    for r in range(16):
        for c in range(4, 8):
            plsc.store_scatter(
                stage, [jnp.full((L,), r, jnp.int32),
                        c * L + lanes], zero16)
    posb[...] = jnp.full((L,), DUMP, jnp.int32)

    def do_table(tab_hbm, ids_hbm, out_hbm):
        pltpu.sync_copy(ids_hbm, idb)
        lo128 = W0 * 128
        hi128 = (W0 + NWIN) * 128

        def a_body(g, cnt):
            ids16 = idb[pl.ds(g * L, L)]
            m = (ids16 >= lo128) & (ids16 < hi128)
            plsc.store_compressed(moff.at[pl.ds(cnt, L)], ids16, mask=m)
            plsc.store_compressed(
                mpos.at[pl.ds(cnt, L)], g * L + lanes, mask=m)
            c = plsc.all_reduce_population_count(m)[0]
            return jnp.minimum(cnt + c, MCAP - L)

        cnt = lax.fori_loop(0, B // L, a_body, jnp.int32(0))
        ngroups = (cnt + L - 1) // L

        def fire_window(widx, slot):
            woff = jnp.minimum(W0 + widx, LASTWIN) * 128
            woff = pl.multiple_of(woff, 128)
            return pltpu.async_copy(
                tab_hbm.at[:, pl.ds(woff, 128)], wbufs[slot], wsems[slot])

        for s in range(4):
            fire_window(s, s)

        def flush_stage(ss):
            pv = posb[...]
            pltpu.async_copy(stage, out_hbm.at[pv], ssem).wait()
            posb[...] = jnp.full((L,), DUMP, jnp.int32)
            return ss

        def process_window(widx, k, ss):
            g = W0 + widx

            def scan(gi, hcnt):
                i16 = moff[pl.ds(gi * L, L)]
                p16 = mpos[pl.ds(gi * L, L)]
                valid = (gi * L + lanes) < cnt
                m = ((i16 >> 7) == g) & valid
                plsc.store_compressed(hid.at[pl.ds(hcnt, L)], i16, mask=m)
                plsc.store_compressed(hpo.at[pl.ds(hcnt, L)], p16, mask=m)
                c = plsc.all_reduce_population_count(m)[0]
                return jnp.minimum(hcnt + c, HCAP - L)

            hcnt = lax.fori_loop(0, ngroups, scan, jnp.int32(0))

            def per_id(i, ss):
                idv = hid[pl.ds(i, L)][0]
                pos = hpo[pl.ds(i, L)][0]
                xs = jnp.full((L,), idv & 127, jnp.int32)
                slot = ss % 16
                rows16 = jnp.full((L,), slot, jnp.int32)
                for c in range(4):
                    vals = plsc.load_gather(wbufs[k], [c * L + lanes, xs])
                    plsc.store_scatter(stage, [rows16, c * L + lanes], vals)
                plsc.store_scatter(
                    posb, [rows16], jnp.full((L,), pos, jnp.int32),
                    mask=lane0)
                return lax.cond(slot == 15, flush_stage,
                                lambda s: s, ss) + 1

            return lax.fori_loop(0, hcnt, per_id, ss)

        def w_body(w4, ss):
            for k in range(4):
                widx = w4 * 4 + k
                pltpu.make_async_copy(
                    tab_hbm.at[:, pl.ds(0, 128)], wbufs[k],
                    wsems[k]).wait()
                ss = lax.cond(
                    widx < NWIN,
                    lambda s, widx=widx, k=k: process_window(widx, k, s),
                    lambda s: s, ss)
                fire_window(widx + 4, k)
            return ss

        nouter = (NWIN + 3) // 4
        ss = lax.fori_loop(0, nouter, w_body, jnp.int32(0))
        for k in range(4):
            pltpu.make_async_copy(
                tab_hbm.at[:, pl.ds(0, 128)], wbufs[k], wsems[k]).wait()
        flush_stage(ss)

    do_table(pu_hbm, uids_hbm, ug_hbm)
    do_table(pv_hbm, iids_hbm, vg_hbm)


def _dot_body(u_ref, v_ref, w_ref, b_ref, o_ref):
    h = u_ref[...] * v_ref[...]
    o_ref[...] = lax.dot_general(
        h, w_ref[...], (((1,), (0,)), ((), ())),
        preferred_element_type=jnp.float32) + b_ref[...]


_dot_tc = pl.pallas_call(
    _dot_body,
    grid=(16,),
    in_specs=[
        pl.BlockSpec((1024, 128), lambda i: (i, 0)),
        pl.BlockSpec((1024, 128), lambda i: (i, 0)),
        pl.BlockSpec((128, 1), lambda i: (0, 0)),
        pl.BlockSpec((1, 1), lambda i: (0, 0)),
    ],
    out_specs=pl.BlockSpec((1024, 1), lambda i: (i, 0)),
    out_shape=jax.ShapeDtypeStruct((B, 1), jnp.float32),
)


def kernel(user_ids, item_ids, user_table, item_table, W, b):
    uids = user_ids.astype(jnp.int32)
    iids = item_ids.astype(jnp.int32)
    ug, vg = _sweep_sc(uids, iids, user_table.T, item_table.T)
    wpad = jnp.zeros((128, 1), jnp.float32).at[:D, 0].set(W[:, 0])
    out2 = _dot_tc(ug, vg, wpad, b.reshape(1, 1))
    return out2[:, 0]


# speedup vs baseline: 1.1498x; 1.1498x over previous
import functools

import jax
import jax.numpy as jnp
from jax import lax
from jax.experimental import pallas as pl
from jax.experimental.pallas import tpu as pltpu
from jax.experimental.pallas import tpu_sc as plsc

B = 16384
D = 64
L = 16
NC = 2
NS = 16
NW = NC * NS
NWINDOWS = 7813
LASTWIN = NWINDOWS - 1
MCAP = 2048
HCAP = 64
DUMP = B
ROWS = B + 128

_mesh = plsc.VectorSubcoreMesh(core_axis_name="c", subcore_axis_name="s")


@functools.partial(
    pl.kernel,
    mesh=_mesh,
    compiler_params=pltpu.CompilerParams(
        needs_layout_passes=False, use_tc_tiling_on_sc=True),
    out_type=(jax.ShapeDtypeStruct((ROWS, 128), jnp.float32),
              jax.ShapeDtypeStruct((ROWS, 128), jnp.float32)),
    scratch_types=[
        pltpu.VMEM((B,), jnp.int32),
        pltpu.VMEM((MCAP,), jnp.int32),
        pltpu.VMEM((MCAP,), jnp.int32),
        pltpu.VMEM((HCAP,), jnp.int32),
        pltpu.VMEM((HCAP,), jnp.int32),
        pltpu.VMEM((256,), jnp.int32),
        pltpu.VMEM((256,), jnp.int32),
        pltpu.VMEM((D, 128), jnp.float32),
        pltpu.VMEM((D, 128), jnp.float32),
        pltpu.VMEM((D, 128), jnp.float32),
        pltpu.VMEM((D, 128), jnp.float32),
        pltpu.VMEM((16, 128), jnp.float32),
        pltpu.VMEM((16,), jnp.int32),
        pltpu.SemaphoreType.DMA,
        pltpu.SemaphoreType.DMA,
        pltpu.SemaphoreType.DMA,
        pltpu.SemaphoreType.DMA,
        pltpu.SemaphoreType.DMA,
    ],
)
def _sweep_sc(uids_hbm, iids_hbm, pu_hbm, pv_hbm, ug_hbm, vg_hbm,
              idb, moff, mpos, hid, hpo, soff, spos, w0b, w1b, w2b, w3b,
              stage, posb, sem0, sem1, sem2, sem3, ssem):
    wid = lax.axis_index("s") * NC + lax.axis_index("c")
    W0 = jnp.where(wid < 5, 245 * wid, 244 * wid + 5)
    NWIN = jnp.where(wid < 5, 245, 244)
    lanes = lax.iota(jnp.int32, L)
    lane0 = lanes == 0
    wbufs = (w0b, w1b, w2b, w3b)
    wsems = (sem0, sem1, sem2, sem3)

    zero16 = jnp.zeros((L,), jnp.float32)
    for r in range(16):
        for c in range(4, 8):
            plsc.store_scatter(
                stage, [jnp.full((L,), r, jnp.int32),
                        c * L + lanes], zero16)
    posb[...] = jnp.full((L,), DUMP, jnp.int32)

    def do_table(tab_hbm, ids_hbm, out_hbm):
        pltpu.sync_copy(ids_hbm, idb)
        lo128 = W0 * 128
        hi128 = (W0 + NWIN) * 128

        def a_body(g, cnt):
            ids16 = idb[pl.ds(g * L, L)]
            m = (ids16 >= lo128) & (ids16 < hi128)
            plsc.store_compressed(moff.at[pl.ds(cnt, L)], ids16, mask=m)
            plsc.store_compressed(
                mpos.at[pl.ds(cnt, L)], g * L + lanes, mask=m)
            c = plsc.all_reduce_population_count(m)[0]
            return jnp.minimum(cnt + c, MCAP - L)

        cnt = lax.fori_loop(0, B // L, a_body, jnp.int32(0))
        ngroups = (cnt + L - 1) // L

        def fire_window(widx, slot):
            woff = jnp.minimum(W0 + widx, LASTWIN) * 128
            woff = pl.multiple_of(woff, 128)
            return pltpu.async_copy(
                tab_hbm.at[:, pl.ds(woff, 128)], wbufs[slot], wsems[slot])

        for s in range(4):
            fire_window(s, s)

        def flush_stage(ss):
            pv = posb[...]
            pltpu.async_copy(stage, out_hbm.at[pv], ssem).wait()
            posb[...] = jnp.full((L,), DUMP, jnp.int32)
            return ss

        def super_filter(w4):
            sw0 = W0 + w4 * 4

            def sscan(gi, sc):
                i16 = moff[pl.ds(gi * L, L)]
                p16 = mpos[pl.ds(gi * L, L)]
                valid = (gi * L + lanes) < cnt
                wg = i16 >> 7
                m = (wg >= sw0) & (wg < sw0 + 16) & valid
                plsc.store_compressed(soff.at[pl.ds(sc, L)], i16, mask=m)
                plsc.store_compressed(spos.at[pl.ds(sc, L)], p16, mask=m)
                c = plsc.all_reduce_population_count(m)[0]
                return jnp.minimum(sc + c, 256 - L)

            return lax.fori_loop(0, ngroups, sscan, jnp.int32(0))

        def process_window(widx, k, ss, scnt):
            g = W0 + widx

            def scan(gi, hcnt):
                i16 = soff[pl.ds(gi * L, L)]
                p16 = spos[pl.ds(gi * L, L)]
                valid = (gi * L + lanes) < scnt
                m = ((i16 >> 7) == g) & valid
                plsc.store_compressed(hid.at[pl.ds(hcnt, L)], i16, mask=m)
                plsc.store_compressed(hpo.at[pl.ds(hcnt, L)], p16, mask=m)
                c = plsc.all_reduce_population_count(m)[0]
                return jnp.minimum(hcnt + c, HCAP - L)

            sgroups = (scnt + L - 1) // L
            hcnt = lax.fori_loop(0, sgroups, scan, jnp.int32(0))

            def per_id(i, ss):
                idv = hid[pl.ds(i, L)][0]
                pos = hpo[pl.ds(i, L)][0]
                xs = jnp.full((L,), idv & 127, jnp.int32)
                slot = ss % 16
                rows16 = jnp.full((L,), slot, jnp.int32)
                for c in range(4):
                    vals = plsc.load_gather(wbufs[k], [c * L + lanes, xs])
                    plsc.store_scatter(stage, [rows16, c * L + lanes], vals)
                plsc.store_scatter(
                    posb, [rows16], jnp.full((L,), pos, jnp.int32),
                    mask=lane0)
                return lax.cond(slot == 15, flush_stage,
                                lambda s: s, ss) + 1

            return lax.fori_loop(0, hcnt, per_id, ss)

        def w_body(w4, carry):
            ss, scnt = carry
            scnt = lax.cond(w4 % 4 == 0,
                            lambda: super_filter(w4),
                            lambda: scnt)
            for k in range(4):
                widx = w4 * 4 + k
                pltpu.make_async_copy(
                    tab_hbm.at[:, pl.ds(0, 128)], wbufs[k],
                    wsems[k]).wait()
                ss = lax.cond(
                    widx < NWIN,
                    lambda s, widx=widx, k=k: process_window(
                        widx, k, s, scnt),
                    lambda s: s, ss)
                fire_window(widx + 4, k)
            return ss, scnt

        nouter = (NWIN + 3) // 4
        ss, _ = lax.fori_loop(
            0, nouter, w_body, (jnp.int32(0), jnp.int32(0)))
        for k in range(4):
            pltpu.make_async_copy(
                tab_hbm.at[:, pl.ds(0, 128)], wbufs[k], wsems[k]).wait()
        flush_stage(ss)

    do_table(pu_hbm, uids_hbm, ug_hbm)
    do_table(pv_hbm, iids_hbm, vg_hbm)


def _dot_body(u_ref, v_ref, w_ref, b_ref, o_ref):
    h = u_ref[...] * v_ref[...]
    o_ref[...] = lax.dot_general(
        h, w_ref[...], (((1,), (0,)), ((), ())),
        preferred_element_type=jnp.float32) + b_ref[...]


_dot_tc = pl.pallas_call(
    _dot_body,
    grid=(16,),
    in_specs=[
        pl.BlockSpec((1024, 128), lambda i: (i, 0)),
        pl.BlockSpec((1024, 128), lambda i: (i, 0)),
        pl.BlockSpec((128, 1), lambda i: (0, 0)),
        pl.BlockSpec((1, 1), lambda i: (0, 0)),
    ],
    out_specs=pl.BlockSpec((1024, 1), lambda i: (i, 0)),
    out_shape=jax.ShapeDtypeStruct((B, 1), jnp.float32),
)


def kernel(user_ids, item_ids, user_table, item_table, W, b):
    uids = user_ids.astype(jnp.int32)
    iids = item_ids.astype(jnp.int32)
    ug, vg = _sweep_sc(uids, iids, user_table.T, item_table.T)
    wpad = jnp.zeros((128, 1), jnp.float32).at[:D, 0].set(W[:, 0])
    out2 = _dot_tc(ug, vg, wpad, b.reshape(1, 1))
    return out2[:, 0]


# speedup vs baseline: 1.1701x; 1.0176x over previous
import functools

import jax
import jax.numpy as jnp
from jax import lax
from jax.experimental import pallas as pl
from jax.experimental.pallas import tpu as pltpu
from jax.experimental.pallas import tpu_sc as plsc

B = 16384
D = 64
L = 16
NC = 2
NS = 16
NW = NC * NS
NWINDOWS = 7813
LASTWIN = NWINDOWS - 1
MCAP = 2048
HCAP = 64
DUMP = B
ROWS = B + 128

_mesh = plsc.VectorSubcoreMesh(core_axis_name="c", subcore_axis_name="s")


@functools.partial(
    pl.kernel,
    mesh=_mesh,
    compiler_params=pltpu.CompilerParams(
        needs_layout_passes=False, use_tc_tiling_on_sc=True),
    out_type=(jax.ShapeDtypeStruct((ROWS, 128), jnp.float32),
              jax.ShapeDtypeStruct((ROWS, 128), jnp.float32)),
    scratch_types=[
        pltpu.VMEM((B,), jnp.int32),
        pltpu.VMEM((MCAP,), jnp.int32),
        pltpu.VMEM((MCAP,), jnp.int32),
        pltpu.VMEM((HCAP,), jnp.int32),
        pltpu.VMEM((HCAP,), jnp.int32),
        pltpu.VMEM((256,), jnp.int32),
        pltpu.VMEM((256,), jnp.int32),
        pltpu.VMEM((D, 128), jnp.float32),
        pltpu.VMEM((D, 128), jnp.float32),
        pltpu.VMEM((D, 128), jnp.float32),
        pltpu.VMEM((D, 128), jnp.float32),
        pltpu.VMEM((16, 128), jnp.float32),
        pltpu.VMEM((16,), jnp.int32),
        pltpu.SemaphoreType.DMA,
        pltpu.SemaphoreType.DMA,
        pltpu.SemaphoreType.DMA,
        pltpu.SemaphoreType.DMA,
        pltpu.SemaphoreType.DMA,
    ],
)
def _sweep_sc(uids_hbm, iids_hbm, pu_hbm, pv_hbm, ug_hbm, vg_hbm,
              idb, moff, mpos, hid, hpo, soff, spos, w0b, w1b, w2b, w3b,
              stage, posb, sem0, sem1, sem2, sem3, ssem):
    wid = lax.axis_index("s") * NC + lax.axis_index("c")
    W0 = jnp.where(wid < 5, 245 * wid, 244 * wid + 5)
    NWIN = jnp.where(wid < 5, 245, 244)
    lanes = lax.iota(jnp.int32, L)
    lane0 = lanes == 0
    wbufs = (w0b, w1b, w2b, w3b)
    wsems = (sem0, sem1, sem2, sem3)

    zero16 = jnp.zeros((L,), jnp.float32)
    for r in range(16):
        for c in range(4, 8):
            plsc.store_scatter(
                stage, [jnp.full((L,), r, jnp.int32),
                        c * L + lanes], zero16)
    posb[...] = jnp.full((L,), DUMP, jnp.int32)

    def do_table(tab_hbm, ids_hbm, out_hbm):
        pltpu.sync_copy(ids_hbm, idb)
        lo128 = W0 * 128
        hi128 = (W0 + NWIN) * 128

        def a_body(g, cnt):
            ids16 = idb[pl.ds(g * L, L)]
            m = (ids16 >= lo128) & (ids16 < hi128)
            plsc.store_compressed(moff.at[pl.ds(cnt, L)], ids16, mask=m)
            plsc.store_compressed(
                mpos.at[pl.ds(cnt, L)], g * L + lanes, mask=m)
            c = plsc.all_reduce_population_count(m)[0]
            return jnp.minimum(cnt + c, MCAP - L)

        cnt = lax.fori_loop(0, B // L, a_body, jnp.int32(0))
        ngroups = (cnt + L - 1) // L

        def fire_window(widx, slot):
            woff = jnp.minimum(W0 + widx, LASTWIN) * 128
            woff = pl.multiple_of(woff, 128)
            return pltpu.async_copy(
                tab_hbm.at[:, pl.ds(woff, 128)], wbufs[slot], wsems[slot])

        for s in range(4):
            fire_window(s, s)

        def flush_stage(ss):
            pv = posb[...]
            pltpu.async_copy(stage, out_hbm.at[pv], ssem).wait()
            posb[...] = jnp.full((L,), DUMP, jnp.int32)
            return ss

        def super_filter(w4):
            sw0 = W0 + w4 * 4

            def sscan(gi, sc):
                i16 = moff[pl.ds(gi * L, L)]
                p16 = mpos[pl.ds(gi * L, L)]
                valid = (gi * L + lanes) < cnt
                wg = i16 >> 7
                m = (wg >= sw0) & (wg < sw0 + 16) & valid
                plsc.store_compressed(soff.at[pl.ds(sc, L)], i16, mask=m)
                plsc.store_compressed(spos.at[pl.ds(sc, L)], p16, mask=m)
                c = plsc.all_reduce_population_count(m)[0]
                return jnp.minimum(sc + c, 256 - L)

            return lax.fori_loop(0, ngroups, sscan, jnp.int32(0))

        def process_window(widx, k, ss, scnt):
            g = W0 + widx

            def scan(gi, hcnt):
                i16 = soff[pl.ds(gi * L, L)]
                p16 = spos[pl.ds(gi * L, L)]
                valid = (gi * L + lanes) < scnt
                m = ((i16 >> 7) == g) & valid
                plsc.store_compressed(hid.at[pl.ds(hcnt, L)], i16, mask=m)
                plsc.store_compressed(hpo.at[pl.ds(hcnt, L)], p16, mask=m)
                c = plsc.all_reduce_population_count(m)[0]
                return jnp.minimum(hcnt + c, HCAP - L)

            sgroups = (scnt + L - 1) // L
            hcnt = lax.fori_loop(0, sgroups, scan, jnp.int32(0))

            def per_id(i, ss):
                idv = hid[pl.ds(i, L)][0]
                pos = hpo[pl.ds(i, L)][0]
                xs = jnp.full((L,), idv & 127, jnp.int32)
                slot = ss % 16
                rows16 = jnp.full((L,), slot, jnp.int32)
                for c in range(4):
                    vals = plsc.load_gather(wbufs[k], [c * L + lanes, xs])
                    plsc.store_scatter(stage, [rows16, c * L + lanes], vals)
                plsc.store_scatter(
                    posb, [rows16], jnp.full((L,), pos, jnp.int32),
                    mask=lane0)
                return lax.cond(slot == 15, flush_stage,
                                lambda s: s, ss) + 1

            return lax.fori_loop(0, hcnt, per_id, ss)

        def w_body(w4, carry):
            ss, scnt = carry
            scnt = lax.cond(w4 % 4 == 0,
                            lambda: super_filter(w4),
                            lambda: scnt)
            for k in range(4):
                widx = w4 * 4 + k
                pltpu.make_async_copy(
                    tab_hbm.at[:, pl.ds(0, 128)], wbufs[k],
                    wsems[k]).wait()
                ss = lax.cond(
                    widx < NWIN,
                    lambda s, widx=widx, k=k: process_window(
                        widx, k, s, scnt),
                    lambda s: s, ss)
                fire_window(widx + 4, k)
            return ss, scnt

        nouter = (NWIN + 3) // 4
        ss, _ = lax.fori_loop(
            0, nouter, w_body, (jnp.int32(0), jnp.int32(0)))
        for k in range(4):
            pltpu.make_async_copy(
                tab_hbm.at[:, pl.ds(0, 128)], wbufs[k], wsems[k]).wait()
        flush_stage(ss)

    do_table(pu_hbm, uids_hbm, ug_hbm)
    do_table(pv_hbm, iids_hbm, vg_hbm)


def _dot_body(u_ref, v_ref, w_ref, b_ref, o_ref):
    h = u_ref[...] * v_ref[...]
    o_ref[...] = lax.dot_general(
        h, w_ref[...], (((1,), (0,)), ((), ())),
        preferred_element_type=jnp.float32) + b_ref[...]


_dot_tc = pl.pallas_call(
    _dot_body,
    grid=(8,),
    in_specs=[
        pl.BlockSpec((2048, 128), lambda i: (i, 0)),
        pl.BlockSpec((2048, 128), lambda i: (i, 0)),
        pl.BlockSpec((128, 1), lambda i: (0, 0)),
        pl.BlockSpec((1, 1), lambda i: (0, 0)),
    ],
    out_specs=pl.BlockSpec((2048, 1), lambda i: (i, 0)),
    out_shape=jax.ShapeDtypeStruct((B, 1), jnp.float32),
)


def kernel(user_ids, item_ids, user_table, item_table, W, b):
    uids = user_ids.astype(jnp.int32)
    iids = item_ids.astype(jnp.int32)
    ug, vg = _sweep_sc(uids, iids, user_table.T, item_table.T)
    wpad = jnp.zeros((128, 1), jnp.float32).at[:D, 0].set(W[:, 0])
    out2 = _dot_tc(ug, vg, wpad, b.reshape(1, 1))
    return out2[:, 0]


# speedup vs baseline: 1.2789x; 1.0931x over previous
import functools

import jax
import jax.numpy as jnp
from jax import lax
from jax.experimental import pallas as pl
from jax.experimental.pallas import tpu as pltpu
from jax.experimental.pallas import tpu_sc as plsc

B = 16384
D = 64
L = 16
NC = 2
NS = 16
NW = NC * NS
NWINDOWS = 7813
LASTWIN = NWINDOWS - 1
MCAP = 2048
HCAP = 64
DUMP = B
ROWS = B + 128

_mesh = plsc.VectorSubcoreMesh(core_axis_name="c", subcore_axis_name="s")


@functools.partial(
    pl.kernel,
    mesh=_mesh,
    compiler_params=pltpu.CompilerParams(
        needs_layout_passes=False, use_tc_tiling_on_sc=True),
    out_type=(jax.ShapeDtypeStruct((ROWS, 128), jnp.float32),
              jax.ShapeDtypeStruct((ROWS, 128), jnp.float32)),
    scratch_types=[
        pltpu.VMEM((B,), jnp.int32),
        pltpu.VMEM((MCAP,), jnp.int32),
        pltpu.VMEM((MCAP,), jnp.int32),
        pltpu.VMEM((HCAP,), jnp.int32),
        pltpu.VMEM((HCAP,), jnp.int32),
        pltpu.VMEM((256,), jnp.int32),
        pltpu.VMEM((256,), jnp.int32),
        pltpu.VMEM((256,), jnp.int32),
        pltpu.VMEM((272,), jnp.int32),
        pltpu.VMEM((D, 128), jnp.float32),
        pltpu.VMEM((D, 128), jnp.float32),
        pltpu.VMEM((D, 128), jnp.float32),
        pltpu.VMEM((D, 128), jnp.float32),
        pltpu.VMEM((16, 128), jnp.float32),
        pltpu.VMEM((16,), jnp.int32),
        pltpu.SemaphoreType.DMA,
        pltpu.SemaphoreType.DMA,
        pltpu.SemaphoreType.DMA,
        pltpu.SemaphoreType.DMA,
        pltpu.SemaphoreType.DMA,
    ],
)
def _sweep_sc(uids_hbm, iids_hbm, pu_hbm, pv_hbm, ug_hbm, vg_hbm,
              idb, moff, mpos, hid, hpo, soff, spos, wcnts, wlist, w0b, w1b, w2b, w3b,
              stage, posb, sem0, sem1, sem2, sem3, ssem):
    wid = lax.axis_index("s") * NC + lax.axis_index("c")
    W0 = jnp.where(wid < 5, 245 * wid, 244 * wid + 5)
    NWIN = jnp.where(wid < 5, 245, 244)
    lanes = lax.iota(jnp.int32, L)
    lane0 = lanes == 0
    wbufs = (w0b, w1b, w2b, w3b)
    wsems = (sem0, sem1, sem2, sem3)

    zero16 = jnp.zeros((L,), jnp.float32)
    for r in range(16):
        for c in range(4, 8):
            plsc.store_scatter(
                stage, [jnp.full((L,), r, jnp.int32),
                        c * L + lanes], zero16)
    posb[...] = jnp.full((L,), DUMP, jnp.int32)

    def do_table(tab_hbm, ids_hbm, out_hbm):
        pltpu.sync_copy(ids_hbm, idb)
        lo128 = W0 * 128
        hi128 = (W0 + NWIN) * 128

        def a_body(g, cnt):
            ids16 = idb[pl.ds(g * L, L)]
            m = (ids16 >= lo128) & (ids16 < hi128)
            plsc.store_compressed(moff.at[pl.ds(cnt, L)], ids16, mask=m)
            plsc.store_compressed(
                mpos.at[pl.ds(cnt, L)], g * L + lanes, mask=m)
            c = plsc.all_reduce_population_count(m)[0]
            return jnp.minimum(cnt + c, MCAP - L)

        cnt = lax.fori_loop(0, B // L, a_body, jnp.int32(0))
        ngroups = (cnt + L - 1) // L

        zeros16i = jnp.zeros((L,), jnp.int32)
        for i in range(16):
            wcnts[pl.ds(i * L, L)] = zeros16i
        ones16 = jnp.full((L,), 1, jnp.int32)

        def h_body(gi, _):
            i16 = moff[pl.ds(gi * L, L)]
            valid = (gi * L + lanes) < cnt
            plsc.addupdate_scatter(
                wcnts, [(i16 >> 7) - W0], ones16, mask=valid)
            return 0

        lax.fori_loop(0, ngroups, h_body, 0)

        def l_body(gi, wc):
            widx16 = gi * L + lanes
            c16 = wcnts[pl.ds(gi * L, L)]
            m = (c16 > 0) & (widx16 < NWIN)
            plsc.store_compressed(wlist.at[pl.ds(wc, L)], widx16, mask=m)
            return wc + plsc.all_reduce_population_count(m)[0]

        wcount = lax.fori_loop(0, 16, l_body, jnp.int32(0))

        def entry(j):
            jj = jnp.maximum(jnp.minimum(j, wcount - 1), 0)
            return wlist[pl.ds(jj, L)][0]

        def fire_entry(j, slot):
            woff = jnp.minimum(W0 + entry(j), LASTWIN) * 128
            woff = pl.multiple_of(woff, 128)
            return pltpu.async_copy(
                tab_hbm.at[:, pl.ds(woff, 128)], wbufs[slot], wsems[slot])

        for s in range(4):
            fire_entry(jnp.int32(s), s)

        def flush_stage(ss):
            pv = posb[...]
            pltpu.async_copy(stage, out_hbm.at[pv], ssem).wait()
            posb[...] = jnp.full((L,), DUMP, jnp.int32)
            return ss

        def super_filter(w4):
            e0 = w4 * 4
            sw_lo = W0 + entry(e0)
            sw_hi = W0 + entry(e0 + 15)

            def sscan(gi, sc):
                i16 = moff[pl.ds(gi * L, L)]
                p16 = mpos[pl.ds(gi * L, L)]
                valid = (gi * L + lanes) < cnt
                wg = i16 >> 7
                m = (wg >= sw_lo) & (wg <= sw_hi) & valid
                plsc.store_compressed(soff.at[pl.ds(sc, L)], i16, mask=m)
                plsc.store_compressed(spos.at[pl.ds(sc, L)], p16, mask=m)
                c = plsc.all_reduce_population_count(m)[0]
                return jnp.minimum(sc + c, 256 - L)

            return lax.fori_loop(0, ngroups, sscan, jnp.int32(0))

        def process_window(j, k, ss, scnt):
            g = W0 + entry(j)

            def scan(gi, hcnt):
                i16 = soff[pl.ds(gi * L, L)]
                p16 = spos[pl.ds(gi * L, L)]
                valid = (gi * L + lanes) < scnt
                m = ((i16 >> 7) == g) & valid
                plsc.store_compressed(hid.at[pl.ds(hcnt, L)], i16, mask=m)
                plsc.store_compressed(hpo.at[pl.ds(hcnt, L)], p16, mask=m)
                c = plsc.all_reduce_population_count(m)[0]
                return jnp.minimum(hcnt + c, HCAP - L)

            sgroups = (scnt + L - 1) // L
            hcnt = lax.fori_loop(0, sgroups, scan, jnp.int32(0))

            def per_id(i, ss):
                idv = hid[pl.ds(i, L)][0]
                pos = hpo[pl.ds(i, L)][0]
                xs = jnp.full((L,), idv & 127, jnp.int32)
                slot = ss % 16
                rows16 = jnp.full((L,), slot, jnp.int32)
                for c in range(4):
                    vals = plsc.load_gather(wbufs[k], [c * L + lanes, xs])
                    plsc.store_scatter(stage, [rows16, c * L + lanes], vals)
                plsc.store_scatter(
                    posb, [rows16], jnp.full((L,), pos, jnp.int32),
                    mask=lane0)
                return lax.cond(slot == 15, flush_stage,
                                lambda s: s, ss) + 1

            return lax.fori_loop(0, hcnt, per_id, ss)

        def w_body(w4, carry):
            ss, scnt = carry
            scnt = lax.cond(w4 % 4 == 0,
                            lambda: super_filter(w4),
                            lambda: scnt)
            for k in range(4):
                j = w4 * 4 + k
                pltpu.make_async_copy(
                    tab_hbm.at[:, pl.ds(0, 128)], wbufs[k],
                    wsems[k]).wait()
                ss = lax.cond(
                    j < wcount,
                    lambda s, j=j, k=k: process_window(j, k, s, scnt),
                    lambda s: s, ss)
                fire_entry(j + 4, k)
            return ss, scnt

        nouter = (wcount + 3) // 4
        ss, _ = lax.fori_loop(
            0, nouter, w_body, (jnp.int32(0), jnp.int32(0)))
        for k in range(4):
            pltpu.make_async_copy(
                tab_hbm.at[:, pl.ds(0, 128)], wbufs[k], wsems[k]).wait()
        flush_stage(ss)

    do_table(pu_hbm, uids_hbm, ug_hbm)
    do_table(pv_hbm, iids_hbm, vg_hbm)


def _dot_body(u_ref, v_ref, w_ref, b_ref, o_ref):
    h = u_ref[...] * v_ref[...]
    o_ref[...] = lax.dot_general(
        h, w_ref[...], (((1,), (0,)), ((), ())),
        preferred_element_type=jnp.float32) + b_ref[...]


_dot_tc = pl.pallas_call(
    _dot_body,
    grid=(8,),
    in_specs=[
        pl.BlockSpec((2048, 128), lambda i: (i, 0)),
        pl.BlockSpec((2048, 128), lambda i: (i, 0)),
        pl.BlockSpec((128, 1), lambda i: (0, 0)),
        pl.BlockSpec((1, 1), lambda i: (0, 0)),
    ],
    out_specs=pl.BlockSpec((2048, 1), lambda i: (i, 0)),
    out_shape=jax.ShapeDtypeStruct((B, 1), jnp.float32),
)


def kernel(user_ids, item_ids, user_table, item_table, W, b):
    uids = user_ids.astype(jnp.int32)
    iids = item_ids.astype(jnp.int32)
    ug, vg = _sweep_sc(uids, iids, user_table.T, item_table.T)
    wpad = jnp.zeros((128, 1), jnp.float32).at[:D, 0].set(W[:, 0])
    out2 = _dot_tc(ug, vg, wpad, b.reshape(1, 1))
    return out2[:, 0]


# speedup vs baseline: 1.2871x; 1.0064x over previous
import functools

import jax
import jax.numpy as jnp
from jax import lax
from jax.experimental import pallas as pl
from jax.experimental.pallas import tpu as pltpu
from jax.experimental.pallas import tpu_sc as plsc

B = 16384
D = 64
L = 16
NC = 2
NS = 16
NW = NC * NS
NWINDOWS = 7813
LASTWIN = NWINDOWS - 1
MCAP = 2048
HCAP = 64
DUMP = B
ROWS = B + 128

_mesh = plsc.VectorSubcoreMesh(core_axis_name="c", subcore_axis_name="s")


@functools.partial(
    pl.kernel,
    mesh=_mesh,
    compiler_params=pltpu.CompilerParams(
        needs_layout_passes=False, use_tc_tiling_on_sc=True),
    out_type=(jax.ShapeDtypeStruct((ROWS, 128), jnp.float32),
              jax.ShapeDtypeStruct((ROWS, 128), jnp.float32)),
    scratch_types=[
        pltpu.VMEM((B,), jnp.int32),
        pltpu.VMEM((MCAP,), jnp.int32),
        pltpu.VMEM((MCAP,), jnp.int32),
        pltpu.VMEM((HCAP,), jnp.int32),
        pltpu.VMEM((HCAP,), jnp.int32),
        pltpu.VMEM((256,), jnp.int32),
        pltpu.VMEM((256,), jnp.int32),
        pltpu.VMEM((256,), jnp.int32),
        pltpu.VMEM((272,), jnp.int32),
        pltpu.VMEM((D, 128), jnp.float32),
        pltpu.VMEM((D, 128), jnp.float32),
        pltpu.VMEM((D, 128), jnp.float32),
        pltpu.VMEM((D, 128), jnp.float32),
        pltpu.VMEM((16, 128), jnp.float32),
        pltpu.VMEM((16,), jnp.int32),
        pltpu.SemaphoreType.DMA,
        pltpu.SemaphoreType.DMA,
        pltpu.SemaphoreType.DMA,
        pltpu.SemaphoreType.DMA,
        pltpu.SemaphoreType.DMA,
    ],
)
def _sweep_sc(uids_hbm, iids_hbm, pu_hbm, pv_hbm, ug_hbm, vg_hbm,
              idb, moff, mpos, hid, hpo, soff, spos, wcnts, wlist, w0b, w1b, w2b, w3b,
              stage, posb, sem0, sem1, sem2, sem3, ssem):
    wid = lax.axis_index("s") * NC + lax.axis_index("c")
    W0 = jnp.where(wid < 5, 245 * wid, 244 * wid + 5)
    NWIN = jnp.where(wid < 5, 245, 244)
    lanes = lax.iota(jnp.int32, L)
    lane0 = lanes == 0
    wbufs = (w0b, w1b, w2b, w3b)
    wsems = (sem0, sem1, sem2, sem3)

    zero16 = jnp.zeros((L,), jnp.float32)
    for r in range(16):
        for c in range(4, 8):
            plsc.store_scatter(
                stage, [jnp.full((L,), r, jnp.int32),
                        c * L + lanes], zero16)
    posb[...] = jnp.full((L,), DUMP, jnp.int32)

    def do_table(tab_hbm, ids_hbm, out_hbm):
        pltpu.sync_copy(ids_hbm, idb)
        lo128 = W0 * 128
        hi128 = (W0 + NWIN) * 128

        def a_body(g, cnt):
            ids16 = idb[pl.ds(g * L, L)]
            m = (ids16 >= lo128) & (ids16 < hi128)
            plsc.store_compressed(moff.at[pl.ds(cnt, L)], ids16, mask=m)
            plsc.store_compressed(
                mpos.at[pl.ds(cnt, L)], g * L + lanes, mask=m)
            c = plsc.all_reduce_population_count(m)[0]
            return jnp.minimum(cnt + c, MCAP - L)

        cnt = lax.fori_loop(0, B // L, a_body, jnp.int32(0))
        ngroups = (cnt + L - 1) // L

        zeros16i = jnp.zeros((L,), jnp.int32)
        for i in range(16):
            wcnts[pl.ds(i * L, L)] = zeros16i
        ones16 = jnp.full((L,), 1, jnp.int32)

        def h_body(gi, _):
            i16 = moff[pl.ds(gi * L, L)]
            valid = (gi * L + lanes) < cnt
            plsc.addupdate_scatter(
                wcnts, [(i16 >> 7) - W0], ones16, mask=valid)
            return 0

        lax.fori_loop(0, ngroups, h_body, 0)

        def l_body(gi, wc):
            widx16 = gi * L + lanes
            c16 = wcnts[pl.ds(gi * L, L)]
            m = (c16 > 0) & (widx16 < NWIN)
            plsc.store_compressed(wlist.at[pl.ds(wc, L)], widx16, mask=m)
            return wc + plsc.all_reduce_population_count(m)[0]

        wcount = lax.fori_loop(0, 16, l_body, jnp.int32(0))

        def entry(j):
            jj = jnp.maximum(jnp.minimum(j, wcount - 1), 0)
            return wlist[pl.ds(jj, L)][0]

        def fire_entry(j, slot):
            woff = jnp.minimum(W0 + entry(j), LASTWIN) * 128
            woff = pl.multiple_of(woff, 128)
            return pltpu.async_copy(
                tab_hbm.at[:, pl.ds(woff, 128)], wbufs[slot], wsems[slot])

        for s in range(4):
            fire_entry(jnp.int32(s), s)

        def flush_stage(ss):
            pv = posb[...]
            pltpu.async_copy(stage, out_hbm.at[pv], ssem).wait()
            posb[...] = jnp.full((L,), DUMP, jnp.int32)
            return ss

        def super_filter(w4):
            e0 = w4 * 4
            sw_lo = W0 + entry(e0)
            sw_hi = W0 + entry(e0 + 15)

            def sscan(gi, sc):
                i16 = moff[pl.ds(gi * L, L)]
                p16 = mpos[pl.ds(gi * L, L)]
                valid = (gi * L + lanes) < cnt
                wg = i16 >> 7
                m = (wg >= sw_lo) & (wg <= sw_hi) & valid
                plsc.store_compressed(soff.at[pl.ds(sc, L)], i16, mask=m)
                plsc.store_compressed(spos.at[pl.ds(sc, L)], p16, mask=m)
                c = plsc.all_reduce_population_count(m)[0]
                return jnp.minimum(sc + c, 256 - L)

            return lax.fori_loop(0, ngroups, sscan, jnp.int32(0))

        def process_window(j, k, ss, scnt):
            g = W0 + entry(j)

            def scan(gi, hcnt):
                i16 = soff[pl.ds(gi * L, L)]
                p16 = spos[pl.ds(gi * L, L)]
                valid = (gi * L + lanes) < scnt
                m = ((i16 >> 7) == g) & valid
                plsc.store_compressed(hid.at[pl.ds(hcnt, L)], i16, mask=m)
                plsc.store_compressed(hpo.at[pl.ds(hcnt, L)], p16, mask=m)
                c = plsc.all_reduce_population_count(m)[0]
                return jnp.minimum(hcnt + c, HCAP - L)

            sgroups = (scnt + L - 1) // L
            hcnt = lax.fori_loop(0, sgroups, scan, jnp.int32(0))

            def per_id(i, ss):
                idv = hid[pl.ds(i, L)][0]
                pos = hpo[pl.ds(i, L)][0]
                xs = jnp.full((L,), idv & 127, jnp.int32)
                slot = ss % 16
                rows16 = jnp.full((L,), slot, jnp.int32)
                for c in range(4):
                    vals = plsc.load_gather(wbufs[k], [c * L + lanes, xs])
                    plsc.store_scatter(stage, [rows16, c * L + lanes], vals)
                plsc.store_scatter(
                    posb, [rows16], jnp.full((L,), pos, jnp.int32),
                    mask=lane0)
                return lax.cond(slot == 15, flush_stage,
                                lambda s: s, ss) + 1

            return lax.fori_loop(0, hcnt, per_id, ss)

        def w_body(w4, carry):
            ss, scnt = carry
            scnt = lax.cond(w4 % 4 == 0,
                            lambda: super_filter(w4),
                            lambda: scnt)
            for k in range(4):
                j = w4 * 4 + k
                pltpu.make_async_copy(
                    tab_hbm.at[:, pl.ds(0, 128)], wbufs[k],
                    wsems[k]).wait()
                ss = lax.cond(
                    j < wcount,
                    lambda s, j=j, k=k: process_window(j, k, s, scnt),
                    lambda s: s, ss)
                fire_entry(j + 4, k)
            return ss, scnt

        nouter = (wcount + 3) // 4
        ss, _ = lax.fori_loop(
            0, nouter, w_body, (jnp.int32(0), jnp.int32(0)))
        for k in range(4):
            pltpu.make_async_copy(
                tab_hbm.at[:, pl.ds(0, 128)], wbufs[k], wsems[k]).wait()
        flush_stage(ss)

    do_table(pu_hbm, uids_hbm, ug_hbm)
    do_table(pv_hbm, iids_hbm, vg_hbm)


def _dot_body(u_ref, v_ref, w_ref, b_ref, o_ref):
    h = u_ref[...] * v_ref[...]
    o_ref[...] = lax.dot_general(
        h, w_ref[...], (((1,), (0,)), ((), ())),
        preferred_element_type=jnp.float32) + b_ref[...]


_dot_tc = pl.pallas_call(
    _dot_body,
    grid=(4,),
    in_specs=[
        pl.BlockSpec((4096, 128), lambda i: (i, 0)),
        pl.BlockSpec((4096, 128), lambda i: (i, 0)),
        pl.BlockSpec((128, 1), lambda i: (0, 0)),
        pl.BlockSpec((1, 1), lambda i: (0, 0)),
    ],
    out_specs=pl.BlockSpec((4096, 1), lambda i: (i, 0)),
    out_shape=jax.ShapeDtypeStruct((B, 1), jnp.float32),
)


def kernel(user_ids, item_ids, user_table, item_table, W, b):
    uids = user_ids.astype(jnp.int32)
    iids = item_ids.astype(jnp.int32)
    ug, vg = _sweep_sc(uids, iids, user_table.T, item_table.T)
    wpad = jnp.zeros((128, 1), jnp.float32).at[:D, 0].set(W[:, 0])
    out2 = _dot_tc(ug, vg, wpad, b.reshape(1, 1))
    return out2[:, 0]


# speedup vs baseline: 1.3773x; 1.0701x over previous
import functools

import jax
import jax.numpy as jnp
from jax import lax
from jax.experimental import pallas as pl
from jax.experimental.pallas import tpu as pltpu
from jax.experimental.pallas import tpu_sc as plsc

B = 16384
D = 64
L = 16
NC = 2
NS = 16
NW = NC * NS
NWINDOWS = 7813
LASTWIN = NWINDOWS - 1
MCAP = 2048
HCAP = 64
DUMP = B
ROWS = B + 128

_mesh = plsc.VectorSubcoreMesh(core_axis_name="c", subcore_axis_name="s")


@functools.partial(
    pl.kernel,
    mesh=_mesh,
    compiler_params=pltpu.CompilerParams(
        needs_layout_passes=False, use_tc_tiling_on_sc=True),
    out_type=(jax.ShapeDtypeStruct((ROWS, 128), jnp.float32),
              jax.ShapeDtypeStruct((ROWS, 128), jnp.float32)),
    scratch_types=[
        pltpu.VMEM((B,), jnp.int32),
        pltpu.VMEM((MCAP,), jnp.int32),
        pltpu.VMEM((MCAP,), jnp.int32),
        pltpu.VMEM((HCAP,), jnp.int32),
        pltpu.VMEM((HCAP,), jnp.int32),
        pltpu.VMEM((256,), jnp.int32),
        pltpu.VMEM((256,), jnp.int32),
        pltpu.VMEM((256,), jnp.int32),
        pltpu.VMEM((272,), jnp.int32),
        pltpu.VMEM((D, 128), jnp.float32),
        pltpu.VMEM((D, 128), jnp.float32),
        pltpu.VMEM((D, 128), jnp.float32),
        pltpu.VMEM((D, 128), jnp.float32),
        pltpu.VMEM((D, 128), jnp.float32),
        pltpu.VMEM((D, 128), jnp.float32),
        pltpu.VMEM((16, 128), jnp.float32),
        pltpu.VMEM((16,), jnp.int32),
        pltpu.SemaphoreType.DMA,
        pltpu.SemaphoreType.DMA,
        pltpu.SemaphoreType.DMA,
        pltpu.SemaphoreType.DMA,
        pltpu.SemaphoreType.DMA,
        pltpu.SemaphoreType.DMA,
        pltpu.SemaphoreType.DMA,
    ],
)
def _sweep_sc(uids_hbm, iids_hbm, pu_hbm, pv_hbm, ug_hbm, vg_hbm,
              idb, moff, mpos, hid, hpo, soff, spos, wcnts, wlist, w0b, w1b, w2b, w3b, w4b, w5b,
              stage, posb, sem0, sem1, sem2, sem3, sem4, sem5, ssem):
    wid = lax.axis_index("s") * NC + lax.axis_index("c")
    W0 = jnp.where(wid < 5, 245 * wid, 244 * wid + 5)
    NWIN = jnp.where(wid < 5, 245, 244)
    lanes = lax.iota(jnp.int32, L)
    lane0 = lanes == 0
    wbufs = (w0b, w1b, w2b, w3b, w4b, w5b)
    wsems = (sem0, sem1, sem2, sem3, sem4, sem5)

    zero16 = jnp.zeros((L,), jnp.float32)
    for r in range(16):
        for c in range(4, 8):
            plsc.store_scatter(
                stage, [jnp.full((L,), r, jnp.int32),
                        c * L + lanes], zero16)
    posb[...] = jnp.full((L,), DUMP, jnp.int32)

    def do_table(tab_hbm, ids_hbm, out_hbm):
        pltpu.sync_copy(ids_hbm, idb)
        lo128 = W0 * 128
        hi128 = (W0 + NWIN) * 128

        def a_body(g, cnt):
            ids16 = idb[pl.ds(g * L, L)]
            m = (ids16 >= lo128) & (ids16 < hi128)
            plsc.store_compressed(moff.at[pl.ds(cnt, L)], ids16, mask=m)
            plsc.store_compressed(
                mpos.at[pl.ds(cnt, L)], g * L + lanes, mask=m)
            c = plsc.all_reduce_population_count(m)[0]
            return jnp.minimum(cnt + c, MCAP - L)

        cnt = lax.fori_loop(0, B // L, a_body, jnp.int32(0))
        ngroups = (cnt + L - 1) // L

        zeros16i = jnp.zeros((L,), jnp.int32)
        for i in range(16):
            wcnts[pl.ds(i * L, L)] = zeros16i
        ones16 = jnp.full((L,), 1, jnp.int32)

        def h_body(gi, _):
            i16 = moff[pl.ds(gi * L, L)]
            valid = (gi * L + lanes) < cnt
            plsc.addupdate_scatter(
                wcnts, [(i16 >> 7) - W0], ones16, mask=valid)
            return 0

        lax.fori_loop(0, ngroups, h_body, 0)

        def l_body(gi, wc):
            widx16 = gi * L + lanes
            c16 = wcnts[pl.ds(gi * L, L)]
            m = (c16 > 0) & (widx16 < NWIN)
            plsc.store_compressed(wlist.at[pl.ds(wc, L)], widx16, mask=m)
            return wc + plsc.all_reduce_population_count(m)[0]

        wcount = lax.fori_loop(0, 16, l_body, jnp.int32(0))

        def entry(j):
            jj = jnp.maximum(jnp.minimum(j, wcount - 1), 0)
            return wlist[pl.ds(jj, L)][0]

        def fire_entry(j, slot):
            woff = jnp.minimum(W0 + entry(j), LASTWIN) * 128
            woff = pl.multiple_of(woff, 128)
            return pltpu.async_copy(
                tab_hbm.at[:, pl.ds(woff, 128)], wbufs[slot], wsems[slot])

        for s in range(6):
            fire_entry(jnp.int32(s), s)

        def flush_stage(ss):
            pv = posb[...]
            pltpu.async_copy(stage, out_hbm.at[pv], ssem).wait()
            posb[...] = jnp.full((L,), DUMP, jnp.int32)
            return ss

        def super_filter(w4):
            e0 = w4 * 6
            sw_lo = W0 + entry(e0)
            sw_hi = W0 + entry(e0 + 17)

            def sscan(gi, sc):
                i16 = moff[pl.ds(gi * L, L)]
                p16 = mpos[pl.ds(gi * L, L)]
                valid = (gi * L + lanes) < cnt
                wg = i16 >> 7
                m = (wg >= sw_lo) & (wg <= sw_hi) & valid
                plsc.store_compressed(soff.at[pl.ds(sc, L)], i16, mask=m)
                plsc.store_compressed(spos.at[pl.ds(sc, L)], p16, mask=m)
                c = plsc.all_reduce_population_count(m)[0]
                return jnp.minimum(sc + c, 256 - L)

            return lax.fori_loop(0, ngroups, sscan, jnp.int32(0))

        def process_window(j, k, ss, scnt):
            g = W0 + entry(j)

            def scan(gi, hcnt):
                i16 = soff[pl.ds(gi * L, L)]
                p16 = spos[pl.ds(gi * L, L)]
                valid = (gi * L + lanes) < scnt
                m = ((i16 >> 7) == g) & valid
                plsc.store_compressed(hid.at[pl.ds(hcnt, L)], i16, mask=m)
                plsc.store_compressed(hpo.at[pl.ds(hcnt, L)], p16, mask=m)
                c = plsc.all_reduce_population_count(m)[0]
                return jnp.minimum(hcnt + c, HCAP - L)

            sgroups = (scnt + L - 1) // L
            hcnt = lax.fori_loop(0, sgroups, scan, jnp.int32(0))

            def per_id(i, ss):
                idv = hid[pl.ds(i, L)][0]
                pos = hpo[pl.ds(i, L)][0]
                xs = jnp.full((L,), idv & 127, jnp.int32)
                slot = ss % 16
                rows16 = jnp.full((L,), slot, jnp.int32)
                for c in range(4):
                    vals = plsc.load_gather(wbufs[k], [c * L + lanes, xs])
                    plsc.store_scatter(stage, [rows16, c * L + lanes], vals)
                plsc.store_scatter(
                    posb, [rows16], jnp.full((L,), pos, jnp.int32),
                    mask=lane0)
                return lax.cond(slot == 15, flush_stage,
                                lambda s: s, ss) + 1

            return lax.fori_loop(0, hcnt, per_id, ss)

        def w_body(w4, carry):
            ss, scnt = carry
            scnt = lax.cond(w4 % 3 == 0,
                            lambda: super_filter(w4),
                            lambda: scnt)
            for k in range(6):
                j = w4 * 6 + k
                pltpu.make_async_copy(
                    tab_hbm.at[:, pl.ds(0, 128)], wbufs[k],
                    wsems[k]).wait()
                ss = lax.cond(
                    j < wcount,
                    lambda s, j=j, k=k: process_window(j, k, s, scnt),
                    lambda s: s, ss)
                fire_entry(j + 6, k)
            return ss, scnt

        nouter = (wcount + 5) // 6
        ss, _ = lax.fori_loop(
            0, nouter, w_body, (jnp.int32(0), jnp.int32(0)))
        for k in range(6):
            pltpu.make_async_copy(
                tab_hbm.at[:, pl.ds(0, 128)], wbufs[k], wsems[k]).wait()
        flush_stage(ss)

    do_table(pu_hbm, uids_hbm, ug_hbm)
    do_table(pv_hbm, iids_hbm, vg_hbm)


def _dot_body(u_ref, v_ref, w_ref, b_ref, o_ref):
    h = u_ref[...] * v_ref[...]
    o_ref[...] = lax.dot_general(
        h, w_ref[...], (((1,), (0,)), ((), ())),
        preferred_element_type=jnp.float32) + b_ref[...]


_dot_tc = pl.pallas_call(
    _dot_body,
    grid=(4,),
    in_specs=[
        pl.BlockSpec((4096, 128), lambda i: (i, 0)),
        pl.BlockSpec((4096, 128), lambda i: (i, 0)),
        pl.BlockSpec((128, 1), lambda i: (0, 0)),
        pl.BlockSpec((1, 1), lambda i: (0, 0)),
    ],
    out_specs=pl.BlockSpec((4096, 1), lambda i: (i, 0)),
    out_shape=jax.ShapeDtypeStruct((B, 1), jnp.float32),
)


def kernel(user_ids, item_ids, user_table, item_table, W, b):
    uids = user_ids.astype(jnp.int32)
    iids = item_ids.astype(jnp.int32)
    ug, vg = _sweep_sc(uids, iids, user_table.T, item_table.T)
    wpad = jnp.zeros((128, 1), jnp.float32).at[:D, 0].set(W[:, 0])
    out2 = _dot_tc(ug, vg, wpad, b.reshape(1, 1))
    return out2[:, 0]
